# baseline (speedup 1.0000x reference)
import functools

import jax
import jax.numpy as jnp
from jax import lax
from jax.experimental import pallas as pl
from jax.experimental.pallas import tpu as pltpu
from jax.experimental.pallas import tpu_sc as plsc

N = 10000
D = 128
E = 320000
NC = 2
NS = 16
K = 128
EPT_RAW = E // NS
NCH = 158
EPT = (NCH + 1) * K
NPAD = 10240
RPT = NPAD // NS


def _sc_body(compute_counts, *refs):
    if compute_counts:
        (tab0, tab1, src0, dst0, src1, dst1,
         out0, out1, cnt0, cnt1,
         idx_s, idx_d, idx_s2, idx_d2, rows, rows2, ones, zerok, cstage,
         acc, cacc, sem, sem2, ssem, csem) = refs
    else:
        (tab0, tab1, src0, dst0, src1, dst1,
         out0, out1,
         idx_s, idx_d, idx_s2, idx_d2, rows, rows2, acc, sem, sem2, ssem) = refs

    c = lax.axis_index("c")
    s = lax.axis_index("s")
    zero16 = jnp.zeros((16,), jnp.float32)
    one16 = jnp.ones((16,), jnp.float32)

    def _zrow(r, carry):
        for j in range(D // 16):
            rows[r, pl.ds(j * 16, 16)] = zero16
            rows2[r, pl.ds(j * 16, 16)] = zero16
        return carry
    lax.fori_loop(0, K, _zrow, 0)
    if compute_counts:
        for j in range(K // 16):
            ones[pl.ds(j * 16, 16)] = one16
            zerok[pl.ds(j * 16, 16)] = zero16
        for j in range(RPT // 16):
            cstage[pl.ds(j * 16, 16)] = zero16

    base = s * RPT
    for q in range(RPT // K):
        pltpu.sync_copy(rows, acc.at[pl.ds(base + q * K, K)])
    if compute_counts:
        pltpu.sync_copy(cstage, cacc.at[pl.ds(base, RPT)])
    plsc.subcore_barrier()

    def _process(tab, src, dst):
        def load_idx(i, isr, idr):
            off = s * EPT + i * K
            pltpu.sync_copy(src.at[pl.ds(off, K)], isr)
            pltpu.sync_copy(dst.at[pl.ds(off, K)], idr)

        def half(i, isr, idr, isr2, idr2, rb, rb2, gsem, gsem2):
            load_idx(i + 1, isr2, idr2)
            pltpu.make_async_copy(rb2, acc.at[idr], ssem).wait()
            if compute_counts:
                pltpu.make_async_copy(ones, cacc.at[idr], csem).wait()
            pltpu.async_copy(tab.at[isr2], rb2, gsem2)
            pltpu.make_async_copy(tab.at[isr], rb, gsem).wait()
            pltpu.async_copy(rb, acc.at[idr], ssem, add=True)
            if compute_counts:
                pltpu.async_copy(ones, cacc.at[idr], csem, add=True)

        n16 = jnp.full((16,), N, jnp.int32)
        for j in range(K // 16):
            idx_d2[pl.ds(j * 16, 16)] = n16
        load_idx(0, idx_s, idx_d)
        pltpu.async_copy(tab.at[idx_s], rows, sem)
        pltpu.async_copy(rows2, acc.at[idx_d2], ssem, add=True)
        if compute_counts:
            pltpu.async_copy(zerok, cacc.at[idx_d2], csem, add=True)

        def _chunk(t, carry):
            half(2 * t, idx_s, idx_d, idx_s2, idx_d2, rows, rows2, sem, sem2)
            half(2 * t + 1, idx_s2, idx_d2, idx_s, idx_d, rows2, rows, sem2, sem)
            return carry
        lax.fori_loop(0, NCH // 2, _chunk, 0)
        pltpu.make_async_copy(rows2, acc.at[idx_d], ssem).wait()
        if compute_counts:
            pltpu.make_async_copy(ones, cacc.at[idx_d], csem).wait()
        pltpu.make_async_copy(tab.at[idx_s], rows, sem).wait()

    @pl.when(c == 0)
    def _():
        _process(tab0, src0, dst0)

    @pl.when(c == 1)
    def _():
        _process(tab1, src1, dst1)

    plsc.subcore_barrier()

    def _writeout(out, cnt_out):
        for q in range(RPT // K):
            r0 = base + q * K
            pltpu.sync_copy(acc.at[pl.ds(r0, K)], rows)
            pltpu.sync_copy(rows, out.at[pl.ds(r0, K)])
        if compute_counts:
            pltpu.sync_copy(cacc.at[pl.ds(base, RPT)], cstage)
            pltpu.sync_copy(cstage, cnt_out.at[pl.ds(base, RPT)])

    @pl.when(c == 0)
    def _():
        _writeout(out0, cnt0 if compute_counts else None)

    @pl.when(c == 1)
    def _():
        _writeout(out1, cnt1 if compute_counts else None)


def _make_sc_agg(compute_counts):
    out_type = [jax.ShapeDtypeStruct((NPAD, D), jnp.float32)] * 2
    if compute_counts:
        out_type += [jax.ShapeDtypeStruct((NPAD,), jnp.float32)] * 2
    scratch = [
        pltpu.VMEM((K,), jnp.int32),
        pltpu.VMEM((K,), jnp.int32),
        pltpu.VMEM((K,), jnp.int32),
        pltpu.VMEM((K,), jnp.int32),
        pltpu.VMEM((K, D), jnp.float32),
        pltpu.VMEM((K, D), jnp.float32),
    ]
    if compute_counts:
        scratch += [
            pltpu.VMEM((K,), jnp.float32),
            pltpu.VMEM((K,), jnp.float32),
            pltpu.VMEM((RPT,), jnp.float32),
        ]
    scratch += [pltpu.VMEM_SHARED((NPAD, D), jnp.float32)]
    if compute_counts:
        scratch += [pltpu.VMEM_SHARED((NPAD,), jnp.float32)]
    scratch += [pltpu.SemaphoreType.DMA] * (4 if compute_counts else 3)
    mesh = plsc.VectorSubcoreMesh(
        core_axis_name="c", subcore_axis_name="s", num_cores=NC, num_subcores=NS)
    return pl.kernel(
        functools.partial(_sc_body, compute_counts),
        out_type=tuple(out_type),
        mesh=mesh,
        scratch_types=tuple(scratch),
    )


_sc_agg_counts = _make_sc_agg(True)
_sc_agg = _make_sc_agg(False)


def _tc_sage_body(relu, agg_ref, cnt_ref, x_ref, wl_ref, wr_ref, b_ref, out_ref):
    inv = 1.0 / jnp.maximum(cnt_ref[...], 1.0)
    mean = agg_ref[...] * inv
    dn = (((1,), (1,)), ((), ()))
    out = (lax.dot_general(mean, wl_ref[...], dn, preferred_element_type=jnp.float32)
           + lax.dot_general(x_ref[...], wr_ref[...], dn, preferred_element_type=jnp.float32)
           + b_ref[...])
    if relu:
        out = jnp.maximum(out, 0.0)
    out_ref[...] = out


def _tc_sage(agg, cnt, x, wl, wr, b, relu):
    bt = 2000
    return pl.pallas_call(
        functools.partial(_tc_sage_body, relu),
        grid=(N // bt,),
        in_specs=[
            pl.BlockSpec((bt, D), lambda i: (i, 0)),
            pl.BlockSpec((bt, 1), lambda i: (i, 0)),
            pl.BlockSpec((bt, D), lambda i: (i, 0)),
            pl.BlockSpec((D, D), lambda i: (0, 0)),
            pl.BlockSpec((D, D), lambda i: (0, 0)),
            pl.BlockSpec((1, D), lambda i: (0, 0)),
        ],
        out_specs=pl.BlockSpec((bt, D), lambda i: (i, 0)),
        out_shape=jax.ShapeDtypeStruct((N, D), jnp.float32),
    )(agg, cnt, x, wl, wr, b)


def _pad_edges(v, fill):
    v = v.astype(jnp.int32).reshape(NS, EPT_RAW)
    v = jnp.pad(v, ((0, 0), (0, EPT - EPT_RAW)), constant_values=fill)
    return v.reshape(NS * EPT)


def kernel(x_author, x_paper, edge_index_writes, edge_index_written_by,
           W1_wp_l, W1_wp_r, b1_wp, W1_pa_l, W1_pa_r, b1_pa,
           W2_wp_l, W2_wp_r, b2_wp, W2_pa_l, W2_pa_r, b2_pa):
    srcw = _pad_edges(edge_index_writes[0], 0)
    dstw = _pad_edges(edge_index_writes[1], N)
    srcb = _pad_edges(edge_index_written_by[0], 0)
    dstb = _pad_edges(edge_index_written_by[1], N)

    aggw, aggb, cntw, cntb = _sc_agg_counts(
        x_author, x_paper, srcw, dstw, srcb, dstb)
    cw = cntw[:N, None]
    cb = cntb[:N, None]

    p1 = _tc_sage(aggw[:N], cw, x_paper, W1_wp_l, W1_wp_r, b1_wp[None, :], True)
    a1 = _tc_sage(aggb[:N], cb, x_author, W1_pa_l, W1_pa_r, b1_pa[None, :], True)

    agg2w, agg2b = _sc_agg(a1, p1, srcw, dstw, srcb, dstb)

    p2 = _tc_sage(agg2w[:N], cw, p1, W2_wp_l, W2_wp_r, b2_wp[None, :], False)
    a2 = _tc_sage(agg2b[:N], cb, a1, W2_pa_l, W2_pa_r, b2_pa[None, :], False)
    return (a2, p2)


# speedup vs baseline: 1.0215x; 1.0215x over previous
import functools

import jax
import jax.numpy as jnp
from jax import lax
from jax.experimental import pallas as pl
from jax.experimental.pallas import tpu as pltpu
from jax.experimental.pallas import tpu_sc as plsc

N = 10000
D = 128
E = 320000
NC = 2
NS = 16
K = 128
EPT_RAW = E // NS
NCH = 158
EPT = (NCH + 1) * K
NPAD = 10240
RPT = NPAD // NS


def _sc_body(compute_counts, *refs):
    if compute_counts:
        (tab0, tab1, src0, dst0, src1, dst1,
         out0, out1, cnt0, cnt1,
         idx_s, idx_d, idx_s2, idx_d2, rows, rows2, ones, zerok, cstage,
         acc, cacc, sem, sem2, ssem, csem) = refs
    else:
        (tab0, tab1, src0, dst0, src1, dst1,
         out0, out1,
         idx_s, idx_d, idx_s2, idx_d2, rows, rows2, acc, sem, sem2, ssem) = refs

    c = lax.axis_index("c")
    s = lax.axis_index("s")
    zero16 = jnp.zeros((16,), jnp.float32)
    one16 = jnp.ones((16,), jnp.float32)

    def _zrow(r, carry):
        for j in range(D // 16):
            rows[r, pl.ds(j * 16, 16)] = zero16
            rows2[r, pl.ds(j * 16, 16)] = zero16
        return carry
    lax.fori_loop(0, K, _zrow, 0)
    if compute_counts:
        for j in range(K // 16):
            ones[pl.ds(j * 16, 16)] = one16
            zerok[pl.ds(j * 16, 16)] = zero16
        for j in range(RPT // 16):
            cstage[pl.ds(j * 16, 16)] = zero16

    base = s * RPT
    for q in range(RPT // K):
        pltpu.sync_copy(rows, acc.at[pl.ds(base + q * K, K)])
    if compute_counts:
        pltpu.sync_copy(cstage, cacc.at[pl.ds(base, RPT)])
    plsc.subcore_barrier()

    def _process(tab, src, dst):
        def load_idx(i, isr, idr):
            off = s * EPT + i * K
            pltpu.sync_copy(src.at[pl.ds(off, K)], isr)
            pltpu.sync_copy(dst.at[pl.ds(off, K)], idr)

        def half(i, isr, idr, isr2, idr2, rb, rb2, gsem, gsem2):
            load_idx(i + 1, isr2, idr2)
            pltpu.make_async_copy(rb2, acc.at[idr], ssem).wait()
            if compute_counts:
                pltpu.make_async_copy(ones, cacc.at[idr], csem).wait()
            pltpu.async_copy(tab.at[isr2], rb2, gsem2)
            pltpu.make_async_copy(tab.at[isr], rb, gsem).wait()
            pltpu.async_copy(rb, acc.at[idr], ssem, add=True)
            if compute_counts:
                pltpu.async_copy(ones, cacc.at[idr], csem, add=True)

        n16 = jnp.full((16,), N, jnp.int32)
        for j in range(K // 16):
            idx_d2[pl.ds(j * 16, 16)] = n16
        load_idx(0, idx_s, idx_d)
        pltpu.async_copy(tab.at[idx_s], rows, sem)
        pltpu.async_copy(rows2, acc.at[idx_d2], ssem, add=True)
        if compute_counts:
            pltpu.async_copy(zerok, cacc.at[idx_d2], csem, add=True)

        def _chunk(t, carry):
            half(2 * t, idx_s, idx_d, idx_s2, idx_d2, rows, rows2, sem, sem2)
            half(2 * t + 1, idx_s2, idx_d2, idx_s, idx_d, rows2, rows, sem2, sem)
            return carry
        lax.fori_loop(0, NCH // 2, _chunk, 0)
        pltpu.make_async_copy(rows2, acc.at[idx_d], ssem).wait()
        if compute_counts:
            pltpu.make_async_copy(ones, cacc.at[idx_d], csem).wait()
        pltpu.make_async_copy(tab.at[idx_s], rows, sem).wait()

    @pl.when(c == 0)
    def _():
        _process(tab0, src0, dst0)

    @pl.when(c == 1)
    def _():
        _process(tab1, src1, dst1)

    plsc.subcore_barrier()

    def _writeout(out, cnt_out):
        for q in range(RPT // K):
            r0 = base + q * K
            pltpu.sync_copy(acc.at[pl.ds(r0, K)], rows)
            pltpu.sync_copy(rows, out.at[pl.ds(r0, K)])
        if compute_counts:
            pltpu.sync_copy(cacc.at[pl.ds(base, RPT)], cstage)
            pltpu.sync_copy(cstage, cnt_out.at[pl.ds(base, RPT)])

    @pl.when(c == 0)
    def _():
        _writeout(out0, cnt0 if compute_counts else None)

    @pl.when(c == 1)
    def _():
        _writeout(out1, cnt1 if compute_counts else None)


def _make_sc_agg(compute_counts):
    out_type = [jax.ShapeDtypeStruct((NPAD, D), jnp.float32)] * 2
    if compute_counts:
        out_type += [jax.ShapeDtypeStruct((NPAD,), jnp.float32)] * 2
    scratch = [
        pltpu.VMEM((K,), jnp.int32),
        pltpu.VMEM((K,), jnp.int32),
        pltpu.VMEM((K,), jnp.int32),
        pltpu.VMEM((K,), jnp.int32),
        pltpu.VMEM((K, D), jnp.float32),
        pltpu.VMEM((K, D), jnp.float32),
    ]
    if compute_counts:
        scratch += [
            pltpu.VMEM((K,), jnp.float32),
            pltpu.VMEM((K,), jnp.float32),
            pltpu.VMEM((RPT,), jnp.float32),
        ]
    scratch += [pltpu.VMEM_SHARED((NPAD, D), jnp.float32)]
    if compute_counts:
        scratch += [pltpu.VMEM_SHARED((NPAD,), jnp.float32)]
    scratch += [pltpu.SemaphoreType.DMA] * (4 if compute_counts else 3)
    mesh = plsc.VectorSubcoreMesh(
        core_axis_name="c", subcore_axis_name="s", num_cores=NC, num_subcores=NS)
    return pl.kernel(
        functools.partial(_sc_body, compute_counts),
        out_type=tuple(out_type),
        mesh=mesh,
        scratch_types=tuple(scratch),
    )


_sc_agg_counts = _make_sc_agg(True)
_sc_agg = _make_sc_agg(False)


def _tc_sage_body(relu, agg_ref, cnt_ref, x_ref, wl_ref, wr_ref, b_ref, out_ref):
    inv = 1.0 / jnp.maximum(cnt_ref[...], 1.0)
    mean = agg_ref[...] * inv
    dn = (((1,), (1,)), ((), ()))
    out = (lax.dot_general(mean, wl_ref[...], dn, preferred_element_type=jnp.float32)
           + lax.dot_general(x_ref[...], wr_ref[...], dn, preferred_element_type=jnp.float32)
           + b_ref[...])
    if relu:
        out = jnp.maximum(out, 0.0)
    out_ref[...] = out


def _tc_sage(agg, cnt, x, wl, wr, b, relu):
    bt = 2000
    return pl.pallas_call(
        functools.partial(_tc_sage_body, relu),
        grid=(N // bt,),
        in_specs=[
            pl.BlockSpec((bt, D), lambda i: (i, 0)),
            pl.BlockSpec((bt, 1), lambda i: (i, 0)),
            pl.BlockSpec((bt, D), lambda i: (i, 0)),
            pl.BlockSpec((D, D), lambda i: (0, 0)),
            pl.BlockSpec((D, D), lambda i: (0, 0)),
            pl.BlockSpec((1, D), lambda i: (0, 0)),
        ],
        out_specs=pl.BlockSpec((bt, D), lambda i: (i, 0)),
        out_shape=jax.ShapeDtypeStruct((N, D), jnp.float32),
    )(agg, cnt, x, wl, wr, b)


def _pad_edges(v, fill):
    v = v.astype(jnp.int32).reshape(NS, EPT_RAW)
    v = jnp.pad(v, ((0, 0), (0, EPT - EPT_RAW)), constant_values=fill)
    return v.reshape(NS * EPT)


def kernel(x_author, x_paper, edge_index_writes, edge_index_written_by,
           W1_wp_l, W1_wp_r, b1_wp, W1_pa_l, W1_pa_r, b1_pa,
           W2_wp_l, W2_wp_r, b2_wp, W2_pa_l, W2_pa_r, b2_pa):
    srcw = _pad_edges(edge_index_writes[0], 0)
    dstw = _pad_edges(edge_index_writes[1], N)
    srcb = _pad_edges(edge_index_written_by[0], 0)
    dstb = _pad_edges(edge_index_written_by[1], N)

    aggw, aggb, cntw, cntb = _sc_agg_counts(
        x_author, x_paper, srcw, dstw, srcb, dstb)
    cw = cntw[:, None]
    cb = cntb[:, None]

    p1 = _tc_sage(aggw, cw, x_paper, W1_wp_l, W1_wp_r, b1_wp[None, :], True)
    a1 = _tc_sage(aggb, cb, x_author, W1_pa_l, W1_pa_r, b1_pa[None, :], True)

    agg2w, agg2b = _sc_agg(a1, p1, srcw, dstw, srcb, dstb)

    p2 = _tc_sage(agg2w, cw, p1, W2_wp_l, W2_wp_r, b2_wp[None, :], False)
    a2 = _tc_sage(agg2b, cb, a1, W2_pa_l, W2_pa_r, b2_pa[None, :], False)
    return (a2, p2)
